# Initial kernel scaffold; baseline (speedup 1.0000x reference)
import jax
import jax.numpy as jnp
from jax.experimental import pallas as pl


def kernel(vectors, x, V, senders, w_emb, w_mlp1, w_mlp2, w_mlp3, w_lin1, w_lin2):
    raise NotImplementedError("write your pallas kernel here")


# speedup vs baseline: 4.3682x; 4.3682x over previous
import math

import numpy as np
import jax
import jax.numpy as jnp
from jax.experimental import pallas as pl

N_NODES = 10000
N_EDGES = 160000
X_DIM = 128
MUL_IN = 16
MUL_OUT = 64
P_ENV = 6
AVG_NEIGH = 16.0

_SH_SL = {0: (0, 1), 1: (1, 4), 2: (4, 9), 3: (9, 16)}
_V_SL = {0: (0, 1), 1: (1, 4), 2: (4, 9)}
_PATHS = {0: [(0, 0), (1, 1), (2, 2)],
          1: [(0, 1), (1, 0), (1, 2), (2, 1), (3, 2)],
          2: [(0, 2), (2, 0), (1, 1), (2, 2), (3, 1)]}


def _cg_complex(j1, j2, j3):
    f = math.factorial
    cg = np.zeros((2 * j1 + 1, 2 * j2 + 1, 2 * j3 + 1))
    for m1 in range(-j1, j1 + 1):
        for m2 in range(-j2, j2 + 1):
            m3 = m1 + m2
            if abs(m3) > j3:
                continue
            pref = (2 * j3 + 1) * f(j3 + j1 - j2) * f(j3 - j1 + j2) * f(j1 + j2 - j3) / f(j1 + j2 + j3 + 1)
            pref *= f(j3 + m3) * f(j3 - m3) * f(j1 - m1) * f(j1 + m1) * f(j2 - m2) * f(j2 + m2)
            pref = math.sqrt(pref)
            kmin = max(0, j2 - j3 - m1, j1 - j3 + m2)
            kmax = min(j1 + j2 - j3, j1 - m1, j2 + m2)
            s = 0.0
            for k in range(kmin, kmax + 1):
                s += (-1) ** k / (f(k) * f(j1 + j2 - j3 - k) * f(j1 - m1 - k) * f(j2 + m2 - k) * f(j3 - j2 + m1 + k) * f(j3 - j1 - m2 + k))
            cg[m1 + j1, m2 + j2, m3 + j3] = pref * s
    return cg


def _real_basis(l):
    A = np.zeros((2 * l + 1, 2 * l + 1), dtype=complex)
    A[l, l] = 1.0
    for m in range(1, l + 1):
        A[l + m, l + m] = (-1) ** m / math.sqrt(2)
        A[l + m, l - m] = 1.0 / math.sqrt(2)
        A[l - m, l - m] = 1j / math.sqrt(2)
        A[l - m, l + m] = -1j * (-1) ** m / math.sqrt(2)
    return A


def _cg_real(l1, l2, l3):
    c = _cg_complex(l1, l2, l3)
    w = np.einsum('ia,jb,kc,abc->ijk', _real_basis(l1), _real_basis(l2), _real_basis(l3).conj(), c)
    wr, wi = np.real(w), np.imag(w)
    return wr if np.linalg.norm(wr) >= np.linalg.norm(wi) else wi


_TERMS = []
_PAIRS = []
for _l3, _ps in _PATHS.items():
    for _p, (_l1, _l2) in enumerate(_ps):
        _cg = _cg_real(_l1, _l2, _l3)
        for _i, _j, _k in np.argwhere(np.abs(_cg) > 1e-12):
            _gi = _SH_SL[_l1][0] + int(_i)
            _gj = _V_SL[_l2][0] + int(_j)
            _TERMS.append((_l3, _p, int(_k), _gi, _gj, float(_cg[_i, _j, _k])))
            if (_gi, _gj) not in _PAIRS:
                _PAIRS.append((_gi, _gj))

_BE = 800


def _kernel_a(vec_ref, x_ref, wemb_ref, wy_ref):
    x = x_ref[...]
    w = jnp.dot(x, wemb_ref[...], preferred_element_type=jnp.float32)
    v = vec_ref[...]
    vx, vy, vz = v[:, 0:1], v[:, 1:2], v[:, 2:3]
    r = jnp.sqrt(vx * vx + vy * vy + vz * vz)
    inv = 1.0 / (r + 1e-9)
    nx, ny, nz = vx * inv, vy * inv, vz * inv
    s3, s5, s15 = math.sqrt(3.0), math.sqrt(5.0), math.sqrt(15.0)
    one = jnp.ones_like(nx)
    ys = [one,
          s3 * ny, s3 * nz, s3 * nx,
          s15 * nx * ny, s15 * ny * nz, 0.5 * s5 * (3 * nz * nz - 1.0),
          s15 * nx * nz, 0.5 * s15 * (nx * nx - ny * ny),
          0.25 * math.sqrt(70.0) * ny * (3 * nx * nx - ny * ny),
          math.sqrt(105.0) * nx * ny * nz,
          0.25 * math.sqrt(42.0) * ny * (5 * nz * nz - 1.0),
          0.5 * math.sqrt(7.0) * nz * (5 * nz * nz - 3.0),
          0.25 * math.sqrt(42.0) * nx * (5 * nz * nz - 1.0),
          0.5 * math.sqrt(105.0) * nz * (nx * nx - ny * ny),
          0.25 * math.sqrt(70.0) * nx * (nx * nx - 3 * ny * ny)]
    wy_ref[...] = jnp.concatenate([w * yi for yi in ys], axis=1)


def _silu(h):
    return h / (1.0 + jnp.exp(-h))


def _kernel_d(x_ref, g_ref, vt_ref, vec_ref, w1_ref, w2_ref, w3_ref,
              wl1_ref, wl2_ref, xout_ref, vout_ref):
    g = g_ref[...]
    vt = vt_ref[...]
    prods = {}
    for (i, j) in _PAIRS:
        prods[(i, j)] = g[:, 16 * i:16 * i + 16] * vt[:, 16 * j:16 * j + 16]
    acc = {}
    for (l3, p, k, i, j, c) in _TERMS:
        key = (l3, p, k)
        t = c * prods[(i, j)]
        acc[key] = acc[key] + t if key in acc else t
    outs0 = jnp.concatenate([acc[(0, p, 0)] for p in range(3)], axis=1)
    x2 = jnp.concatenate([x_ref[...], outs0], axis=1)
    h = _silu(jnp.dot(x2, w1_ref[...], preferred_element_type=jnp.float32))
    h = _silu(jnp.dot(h, w2_ref[...], preferred_element_type=jnp.float32))
    h = jnp.dot(h, w3_ref[...], preferred_element_type=jnp.float32)
    v = vec_ref[...]
    vx, vy, vz = v[:, 0:1], v[:, 1:2], v[:, 2:3]
    d = jnp.sqrt(vx * vx + vy * vy + vz * vz)
    d2 = d * d
    d6 = d2 * d2 * d2
    env = jnp.where(d < 1.0, 1.0 - 28.0 * d6 + 48.0 * d6 * d - 21.0 * d6 * d2, 0.0)
    xout_ref[...] = env * h
    o1 = jnp.concatenate([acc[(1, p, m)] for m in range(3) for p in range(5)], axis=1)
    o2 = jnp.concatenate([acc[(2, p, m)] for m in range(5) for p in range(5)], axis=1)
    v1 = jnp.dot(o1, wl1_ref[...], preferred_element_type=jnp.float32)
    v2 = jnp.dot(o2, wl2_ref[...], preferred_element_type=jnp.float32)
    zeros = jnp.zeros_like(v1[:, 0:64])
    vout_ref[...] = jnp.concatenate([zeros, v1, v2], axis=1)


def _edge_spec(width):
    return pl.BlockSpec((_BE, width), lambda i: (i, 0))


def _full_spec(shape):
    return pl.BlockSpec(shape, lambda i: (0, 0))


def kernel(vectors, x, V, senders, w_emb, w_mlp1, w_mlp2, w_mlp3, w_lin1, w_lin2):
    E = N_EDGES
    grid = (E // _BE,)
    wemb_s = w_emb * (1.0 / math.sqrt(X_DIM))

    wy = pl.pallas_call(
        _kernel_a,
        grid=grid,
        in_specs=[_edge_spec(3), _edge_spec(X_DIM), _full_spec((X_DIM, MUL_IN))],
        out_specs=_edge_spec(256),
        out_shape=jax.ShapeDtypeStruct((E, 256), jnp.float32),
    )(vectors, x, wemb_s)

    agg = jax.ops.segment_sum(wy, senders, num_segments=N_NODES)
    agg = agg * (1.0 / math.sqrt(AVG_NEIGH))
    g = agg[senders]
    vt = V.reshape(E, MUL_IN, 9).transpose(0, 2, 1).reshape(E, 144)

    w1r = jnp.concatenate(
        [w_mlp1[:X_DIM],
         w_mlp1[X_DIM:].reshape(MUL_IN, 3, 64).transpose(1, 0, 2).reshape(48, 64)],
        axis=0) * (1.0 / math.sqrt(X_DIM + 48))
    w2s = w_mlp2 * (1.0 / math.sqrt(64.0))
    w3s = w_mlp3 * (1.0 / math.sqrt(64.0))

    scale = 1.0 / math.sqrt(MUL_IN * 5)
    wl1_r = w_lin1.reshape(MUL_IN, 5, MUL_OUT).transpose(1, 0, 2).reshape(80, MUL_OUT) * scale
    wl2_r = w_lin2.reshape(MUL_IN, 5, MUL_OUT).transpose(1, 0, 2).reshape(80, MUL_OUT) * scale
    eye3 = jnp.eye(3, dtype=jnp.float32)
    eye5 = jnp.eye(5, dtype=jnp.float32)
    WL1 = (wl1_r[None, :, :, None] * eye3[:, None, None, :]).reshape(240, 192)
    WL2 = (wl2_r[None, :, :, None] * eye5[:, None, None, :]).reshape(400, 320)

    x_out, v_out = pl.pallas_call(
        _kernel_d,
        grid=grid,
        in_specs=[_edge_spec(X_DIM), _edge_spec(256), _edge_spec(144), _edge_spec(3),
                  _full_spec((176, 64)), _full_spec((64, 64)), _full_spec((64, 64)),
                  _full_spec((240, 192)), _full_spec((400, 320))],
        out_specs=[_edge_spec(64), _edge_spec(64 + 192 + 320)],
        out_shape=[jax.ShapeDtypeStruct((E, 64), jnp.float32),
                   jax.ShapeDtypeStruct((E, 576), jnp.float32)],
    )(x, g, vt, vectors, w1r, w2s, w3s, WL1, WL2)
    return x_out, v_out
